# Initial kernel scaffold; baseline (speedup 1.0000x reference)
import jax
import jax.numpy as jnp
from jax.experimental import pallas as pl


def kernel(x, edge_index, batch, W1, b1, W2, b2):
    raise NotImplementedError("write your pallas kernel here")


# speedup vs baseline: 8.6761x; 8.6761x over previous
import functools

import jax
import jax.numpy as jnp
from jax import lax
from jax.experimental import pallas as pl
from jax.experimental.pallas import tpu as pltpu
from jax.experimental.pallas import tpu_sc as plsc

N = 10000
NP = 10240
E = 320000
EP = 327680
G = 64
D = 128
NC, NS = 2, 16
NT = NC * NS
EPT = EP // NT
NCH = 80
CH = 128
RPT = NP // NS
RB = 1024
NBLK = NP // RB

_mesh = plsc.VectorSubcoreMesh(core_axis_name="c", subcore_axis_name="s")
_sc_params = pltpu.CompilerParams(needs_layout_passes=False)


@functools.partial(
    pl.kernel,
    out_type=jax.ShapeDtypeStruct((NT, NP), jnp.float32),
    mesh=_mesh,
    scratch_types=[
        pltpu.VMEM((EPT,), jnp.int32),
        pltpu.VMEM((NP,), jnp.float32),
    ],
    compiler_params=_sc_params,
)
def _deg(dst_flat, out, dv, part):
    c = lax.axis_index("c")
    s = lax.axis_index("s")
    wid = c * NS + s
    pltpu.sync_copy(dst_flat.at[wid], dv)

    def zero_body(i, carry):
        part[pl.ds(i * 16, 16)] = jnp.zeros((16,), jnp.float32)
        return carry
    lax.fori_loop(0, NP // 16, zero_body, 0)

    ones = jnp.ones((16,), jnp.float32)

    def add_body(i, carry):
        idx = dv[pl.ds(i * 16, 16)]
        plsc.addupdate_scatter(part, [idx], ones)
        return carry
    lax.fori_loop(0, EPT // 16, add_body, 0)

    pltpu.sync_copy(part, out.at[wid])


@functools.partial(
    pl.kernel,
    out_type=jax.ShapeDtypeStruct((NC, NP, D), jnp.float32),
    mesh=_mesh,
    scratch_types=[
        pltpu.VMEM((NCH, CH), jnp.int32),
        pltpu.VMEM((NCH, CH), jnp.int32),
        pltpu.VMEM((CH, D), jnp.float32),
        pltpu.SemaphoreType.DMA,
        pltpu.VMEM_SHARED((NP, D), jnp.float32),
    ],
    compiler_params=_sc_params,
)
def _prop(hp, srcr, dstr, zer, out, src_v, dst_v, buf, sem, acc):
    c = lax.axis_index("c")
    s = lax.axis_index("s")
    wid = c * NS + s
    rows0 = s * RPT
    pltpu.sync_copy(zer.at[pl.ds(rows0, RPT)], acc.at[pl.ds(rows0, RPT)])
    pltpu.sync_copy(srcr.at[wid], src_v)
    pltpu.sync_copy(dstr.at[wid], dst_v)
    plsc.subcore_barrier()

    def body(j, carry):
        pltpu.async_copy(hp.at[src_v.at[j]], buf, sem).wait()
        pltpu.sync_copy(buf, acc.at[dst_v.at[j]], add=True)
        return carry
    lax.fori_loop(0, NCH, body, 0)

    plsc.subcore_barrier()
    pltpu.sync_copy(acc.at[pl.ds(rows0, RPT)], out.at[c, pl.ds(rows0, RPT)])


def _tc_a_body(degp_ref, x_ref, w1_ref, h1p_ref, dis_ref):
    ones = jnp.ones((NT, 1), jnp.float32)
    deg = lax.dot_general(degp_ref[...], ones, (((0,), (0,)), ((), ()))) + 1.0
    dis = lax.rsqrt(deg)
    h = jnp.dot(x_ref[...], w1_ref[...], preferred_element_type=jnp.float32)
    h1p_ref[...] = h * dis
    dis_ref[...] = jnp.broadcast_to(dis, (RB, D))


_tc_a = pl.pallas_call(
    _tc_a_body,
    grid=(NBLK,),
    in_specs=[
        pl.BlockSpec((NT, RB), lambda i: (0, i)),
        pl.BlockSpec((RB, D), lambda i: (i, 0)),
        pl.BlockSpec((D, D), lambda i: (0, 0)),
    ],
    out_specs=[
        pl.BlockSpec((RB, D), lambda i: (i, 0)),
        pl.BlockSpec((RB, D), lambda i: (i, 0)),
    ],
    out_shape=[
        jax.ShapeDtypeStruct((NP, D), jnp.float32),
        jax.ShapeDtypeStruct((NP, D), jnp.float32),
    ],
)


def _tc_b_body(acc_ref, h1p_ref, dis_ref, b1_ref, w2_ref, out_ref):
    ssum = jnp.sum(acc_ref[...], axis=0) + h1p_ref[...]
    h = jnp.maximum(dis_ref[...] * ssum + b1_ref[...], 0.0)
    out_ref[...] = dis_ref[...] * jnp.dot(
        h, w2_ref[...], preferred_element_type=jnp.float32)


_tc_b = pl.pallas_call(
    _tc_b_body,
    grid=(NBLK,),
    in_specs=[
        pl.BlockSpec((NC, RB, D), lambda i: (0, i, 0)),
        pl.BlockSpec((RB, D), lambda i: (i, 0)),
        pl.BlockSpec((RB, D), lambda i: (i, 0)),
        pl.BlockSpec((1, D), lambda i: (0, 0)),
        pl.BlockSpec((D, D), lambda i: (0, 0)),
    ],
    out_specs=pl.BlockSpec((RB, D), lambda i: (i, 0)),
    out_shape=jax.ShapeDtypeStruct((NP, D), jnp.float32),
)


def _tc_c_body(acc_ref, h2p_ref, dis_ref, b2_ref, batch_ref, out_ref, cnt_ref):
    i = pl.program_id(0)
    conv = dis_ref[...] * (jnp.sum(acc_ref[...], axis=0) + h2p_ref[...]) \
        + b2_ref[...]
    b = batch_ref[0]
    gids = lax.broadcasted_iota(jnp.int32, (G, RB), 0)
    onehot = (b == gids).astype(jnp.float32)
    psum = jnp.dot(onehot, conv, preferred_element_type=jnp.float32)
    pcnt = jnp.sum(onehot, axis=1, keepdims=True)

    @pl.when(i == 0)
    def _():
        out_ref[...] = jnp.zeros((G, D), jnp.float32)
        cnt_ref[...] = jnp.zeros((G, 1), jnp.float32)

    out_ref[...] += psum
    cnt_ref[...] += pcnt

    @pl.when(i == NBLK - 1)
    def _():
        out_ref[...] = out_ref[...] / jnp.maximum(cnt_ref[...], 1.0)


_tc_c = pl.pallas_call(
    _tc_c_body,
    grid=(NBLK,),
    in_specs=[
        pl.BlockSpec((NC, RB, D), lambda i: (0, i, 0)),
        pl.BlockSpec((RB, D), lambda i: (i, 0)),
        pl.BlockSpec((RB, D), lambda i: (i, 0)),
        pl.BlockSpec((1, D), lambda i: (0, 0)),
        pl.BlockSpec((1, 1, RB), lambda i: (i, 0, 0)),
    ],
    out_specs=pl.BlockSpec((G, D), lambda i: (0, 0)),
    out_shape=jax.ShapeDtypeStruct((G, D), jnp.float32),
    scratch_shapes=[pltpu.VMEM((G, 1), jnp.float32)],
)


def kernel(x, edge_index, batch, W1, b1, W2, b2):
    src = edge_index[0]
    dst = edge_index[1]
    pad_e = jnp.full((EP - E,), N, jnp.int32)
    srcr = jnp.concatenate([src, pad_e]).reshape(NT, NCH, CH)
    dstp = jnp.concatenate([dst, pad_e])
    dstr = dstp.reshape(NT, NCH, CH)
    dst_flat = dstp.reshape(NT, EPT)
    x_pad = jnp.pad(x, ((0, NP - N), (0, 0)))
    batch_pad = jnp.concatenate(
        [batch, jnp.full((NP - N,), G, jnp.int32)]).reshape(NBLK, 1, RB)
    zeros_np = jnp.zeros((NP, D), jnp.float32)

    degp = _deg(dst_flat)
    h1p, disb = _tc_a(degp, x_pad, W1)
    acc1 = _prop(h1p, srcr, dstr, zeros_np)
    h2p = _tc_b(acc1, h1p, disb, b1.reshape(1, D), W2)
    acc2 = _prop(h2p, srcr, dstr, zeros_np)
    pooled = _tc_c(acc2, h2p, disb, b2.reshape(1, D), batch_pad)
    return pooled


# speedup vs baseline: 17.8258x; 2.0546x over previous
import functools

import jax
import jax.numpy as jnp
from jax import lax
from jax.experimental import pallas as pl
from jax.experimental.pallas import tpu as pltpu
from jax.experimental.pallas import tpu_sc as plsc

N = 10000
NP = 10240
E = 320000
G = 64
D = 128
NC, NS = 2, 16
NT = NC * NS
NCH = 126
CH = 80
EPT = NCH * CH
EP = EPT * NT
RPT = NP // NS
RB = 1024
NBLK = NP // RB

_mesh = plsc.VectorSubcoreMesh(core_axis_name="c", subcore_axis_name="s")
_sc_params = pltpu.CompilerParams(needs_layout_passes=False)


@functools.partial(
    pl.kernel,
    out_type=jax.ShapeDtypeStruct((NT, NP), jnp.float32),
    mesh=_mesh,
    scratch_types=[
        pltpu.VMEM((EPT,), jnp.int32),
        pltpu.VMEM((NP,), jnp.float32),
    ],
    compiler_params=_sc_params,
)
def _deg(dst_flat, out, dv, part):
    c = lax.axis_index("c")
    s = lax.axis_index("s")
    wid = c * NS + s
    pltpu.sync_copy(dst_flat.at[wid], dv)

    def zero_body(i, carry):
        part[pl.ds(i * 16, 16)] = jnp.zeros((16,), jnp.float32)
        return carry
    lax.fori_loop(0, NP // 16, zero_body, 0)

    ones = jnp.ones((16,), jnp.float32)

    def add_body(i, carry):
        idx = dv[pl.ds(i * 16, 16)]
        plsc.addupdate_scatter(part, [idx], ones)
        return carry
    lax.fori_loop(0, EPT // 16, add_body, 0)

    pltpu.sync_copy(part, out.at[wid])


NBUF = 3
NGRP = NCH // NBUF


@functools.partial(
    pl.kernel,
    out_type=jax.ShapeDtypeStruct((NC, NP, D), jnp.float32),
    mesh=_mesh,
    scratch_types=[
        pltpu.VMEM((NBUF, CH), jnp.int32),
        pltpu.VMEM((NBUF, CH), jnp.int32),
        pltpu.VMEM((CH, D), jnp.float32),
        pltpu.VMEM((CH, D), jnp.float32),
        pltpu.VMEM((CH, D), jnp.float32),
        pltpu.SemaphoreType.DMA((NBUF,)),
        pltpu.SemaphoreType.DMA((NBUF,)),
        pltpu.SemaphoreType.DMA((NBUF,)),
        pltpu.VMEM_SHARED((NP, D), jnp.float32),
    ],
    compiler_params=_sc_params,
)
def _prop(hp, srcr, dstr, out, si, di, b0, b1, b2, isem, gsem, ssem, acc):
    bufs = (b0, b1, b2)
    c = lax.axis_index("c")
    s = lax.axis_index("s")
    wid = c * NS + s
    rows0 = s * RPT

    def zrow(i, carry):
        for k in range(D // 16):
            b0[i, pl.ds(k * 16, 16)] = jnp.zeros((16,), jnp.float32)
        return carry
    lax.fori_loop(0, CH, zrow, 0)
    for r in range(RPT // CH):
        pltpu.sync_copy(b0, acc.at[pl.ds(rows0 + r * CH, CH)])
    plsc.subcore_barrier()

    for b in range(NBUF):
        pltpu.async_copy(srcr.at[wid, b], si.at[b], isem.at[b])
        pltpu.async_copy(dstr.at[wid, b], di.at[b], isem.at[b])

    def group(g, carry):
        for b in range(NBUF):
            j = g * NBUF + b
            pltpu.make_async_copy(srcr.at[wid, j], si.at[b],
                                  isem.at[b]).wait()
            pltpu.make_async_copy(dstr.at[wid, j], di.at[b],
                                  isem.at[b]).wait()
            pltpu.async_copy(hp.at[si.at[b]], bufs[b], gsem.at[b])
        for b in range(NBUF):
            pltpu.make_async_copy(hp.at[si.at[b]], bufs[b],
                                  gsem.at[b]).wait()
            pltpu.async_copy(bufs[b], acc.at[di.at[b]], ssem.at[b],
                             add=True)
        for b in range(NBUF):
            j2 = g * NBUF + b + NBUF
            pltpu.make_async_copy(bufs[b], acc.at[di.at[b]],
                                  ssem.at[b]).wait()

            @pl.when(j2 < NCH)
            def _():
                pltpu.async_copy(srcr.at[wid, j2], si.at[b], isem.at[b])
                pltpu.async_copy(dstr.at[wid, j2], di.at[b], isem.at[b])
        return carry
    lax.fori_loop(0, NGRP, group, 0)

    plsc.subcore_barrier()
    pltpu.sync_copy(acc.at[pl.ds(rows0, RPT)], out.at[c, pl.ds(rows0, RPT)])


def _tc_a_body(degp_ref, x_ref, w1_ref, h1p_ref, dis_ref):
    ones = jnp.ones((NT, 1), jnp.float32)
    deg = lax.dot_general(degp_ref[...], ones, (((0,), (0,)), ((), ()))) + 1.0
    dis = lax.rsqrt(deg)
    h = jnp.dot(x_ref[...], w1_ref[...], preferred_element_type=jnp.float32)
    h1p_ref[...] = h * dis
    dis_ref[...] = jnp.broadcast_to(dis, (RB, D))


_tc_a = pl.pallas_call(
    _tc_a_body,
    grid=(NBLK,),
    in_specs=[
        pl.BlockSpec((NT, RB), lambda i: (0, i)),
        pl.BlockSpec((RB, D), lambda i: (i, 0)),
        pl.BlockSpec((D, D), lambda i: (0, 0)),
    ],
    out_specs=[
        pl.BlockSpec((RB, D), lambda i: (i, 0)),
        pl.BlockSpec((RB, D), lambda i: (i, 0)),
    ],
    out_shape=[
        jax.ShapeDtypeStruct((NP, D), jnp.float32),
        jax.ShapeDtypeStruct((NP, D), jnp.float32),
    ],
)


def _tc_b_body(acc_ref, h1p_ref, dis_ref, b1_ref, w2_ref, out_ref):
    ssum = jnp.sum(acc_ref[...], axis=0) + h1p_ref[...]
    h = jnp.maximum(dis_ref[...] * ssum + b1_ref[...], 0.0)
    out_ref[...] = dis_ref[...] * jnp.dot(
        h, w2_ref[...], preferred_element_type=jnp.float32)


_tc_b = pl.pallas_call(
    _tc_b_body,
    grid=(NBLK,),
    in_specs=[
        pl.BlockSpec((NC, RB, D), lambda i: (0, i, 0)),
        pl.BlockSpec((RB, D), lambda i: (i, 0)),
        pl.BlockSpec((RB, D), lambda i: (i, 0)),
        pl.BlockSpec((1, D), lambda i: (0, 0)),
        pl.BlockSpec((D, D), lambda i: (0, 0)),
    ],
    out_specs=pl.BlockSpec((RB, D), lambda i: (i, 0)),
    out_shape=jax.ShapeDtypeStruct((NP, D), jnp.float32),
)


def _tc_c_body(acc_ref, h2p_ref, dis_ref, b2_ref, batch_ref, out_ref, cnt_ref):
    i = pl.program_id(0)
    conv = dis_ref[...] * (jnp.sum(acc_ref[...], axis=0) + h2p_ref[...]) \
        + b2_ref[...]
    b = batch_ref[0]
    gids = lax.broadcasted_iota(jnp.int32, (G, RB), 0)
    onehot = (b == gids).astype(jnp.float32)
    psum = jnp.dot(onehot, conv, preferred_element_type=jnp.float32)
    pcnt = jnp.sum(onehot, axis=1, keepdims=True)

    @pl.when(i == 0)
    def _():
        out_ref[...] = jnp.zeros((G, D), jnp.float32)
        cnt_ref[...] = jnp.zeros((G, 1), jnp.float32)

    out_ref[...] += psum
    cnt_ref[...] += pcnt

    @pl.when(i == NBLK - 1)
    def _():
        out_ref[...] = out_ref[...] / jnp.maximum(cnt_ref[...], 1.0)


_tc_c = pl.pallas_call(
    _tc_c_body,
    grid=(NBLK,),
    in_specs=[
        pl.BlockSpec((NC, RB, D), lambda i: (0, i, 0)),
        pl.BlockSpec((RB, D), lambda i: (i, 0)),
        pl.BlockSpec((RB, D), lambda i: (i, 0)),
        pl.BlockSpec((1, D), lambda i: (0, 0)),
        pl.BlockSpec((1, 1, RB), lambda i: (i, 0, 0)),
    ],
    out_specs=pl.BlockSpec((G, D), lambda i: (0, 0)),
    out_shape=jax.ShapeDtypeStruct((G, D), jnp.float32),
    scratch_shapes=[pltpu.VMEM((G, 1), jnp.float32)],
)


def kernel(x, edge_index, batch, W1, b1, W2, b2):
    src = edge_index[0]
    dst = edge_index[1]
    pad_e = jnp.full((EP - E,), N, jnp.int32)
    srcr = jnp.concatenate([src, pad_e]).reshape(NT, NCH, CH)
    dstp = jnp.concatenate([dst, pad_e])
    dstr = dstp.reshape(NT, NCH, CH)
    dst_flat = dstp.reshape(NT, EPT)
    x_pad = jnp.pad(x, ((0, NP - N), (0, 0)))
    batch_pad = jnp.concatenate(
        [batch, jnp.full((NP - N,), G, jnp.int32)]).reshape(NBLK, 1, RB)

    degp = _deg(dst_flat)
    h1p, disb = _tc_a(degp, x_pad, W1)
    acc1 = _prop(h1p, srcr, dstr)
    h2p = _tc_b(acc1, h1p, disb, b1.reshape(1, D), W2)
    acc2 = _prop(h2p, srcr, dstr)
    pooled = _tc_c(acc2, h2p, disb, b2.reshape(1, D), batch_pad)
    return pooled
